# baseline (speedup 1.0000x reference)
import functools

import jax
import jax.numpy as jnp
from jax import lax
from jax.experimental import pallas as pl
from jax.experimental.pallas import tpu as pltpu
from jax.experimental.pallas import tpu_sc as plsc

N = 10000
E = 160000
D = 256
NUM_CLASSES = 2

NCORES = 2
NSUB = 16
NWORK = NCORES * NSUB
C = 128
CH = 40
EW = C * CH
EPAD = EW * NWORK
NP = 10240
RPT = NP // NSUB


def _make_edge_pass(width, const_table):
  mesh = plsc.VectorSubcoreMesh(core_axis_name="c", subcore_axis_name="s")

  nbuf = 4

  @functools.partial(
      pl.kernel,
      out_type=jax.ShapeDtypeStruct((NCORES, NP, width), jnp.float32),
      mesh=mesh,
      scratch_types=[
          pltpu.VMEM((CH, C), jnp.int32),
          pltpu.VMEM((CH, C), jnp.int32),
          [pltpu.VMEM((C, width), jnp.float32) for _ in range(nbuf)],
          pltpu.VMEM_SHARED((NP, width), jnp.float32),
          [pltpu.SemaphoreType.DMA for _ in range(nbuf)],
      ],
      compiler_params=pltpu.CompilerParams(use_tc_tiling_on_sc=False),
  )
  def edge_pass(src_hbm, dst_hbm, table_hbm, zeros_hbm, out_hbm,
                src_v, dst_v, rows, acc_sh, sems):
    cid = lax.axis_index("c")
    sid = lax.axis_index("s")
    wid = cid * NSUB + sid
    pltpu.sync_copy(zeros_hbm.at[pl.ds(sid * RPT, RPT)],
                    acc_sh.at[pl.ds(sid * RPT, RPT)])
    pltpu.sync_copy(src_hbm.at[pl.ds(wid * CH, CH)], src_v)
    pltpu.sync_copy(dst_hbm.at[pl.ds(wid * CH, CH)], dst_v)
    plsc.subcore_barrier()

    def wait_gather(b, j):
      pltpu.make_async_copy(table_hbm.at[src_v.at[j]], rows[b],
                            sems[b]).wait()

    def wait_scatter(b, j):
      pltpu.make_async_copy(rows[b], acc_sh.at[dst_v.at[j]],
                            sems[b]).wait()

    if const_table:
      pltpu.sync_copy(table_hbm, rows[0])
      for b in range(nbuf):
        pltpu.async_copy(rows[0], acc_sh.at[dst_v.at[b]], sems[b],
                         add=True)

      def step(jj, carry):
        j = jj * nbuf
        for b in range(nbuf):
          pltpu.make_async_copy(rows[0], acc_sh.at[dst_v.at[j + b]],
                                sems[b]).wait()
          pltpu.async_copy(rows[0], acc_sh.at[dst_v.at[j + b]], sems[b],
                           add=True)
        return carry

      lax.fori_loop(1, CH // nbuf, step, 0, unroll=False)
      for b in range(nbuf):
        pltpu.make_async_copy(rows[0], acc_sh.at[dst_v.at[CH - nbuf + b]],
                              sems[b]).wait()
    else:
      for b in range(nbuf):
        pltpu.async_copy(table_hbm.at[src_v.at[b]], rows[b], sems[b])

      def step(jj, carry):
        j = jj * nbuf
        for b in range(nbuf):
          wait_gather(b, j + b)
          pltpu.async_copy(rows[b], acc_sh.at[dst_v.at[j + b]], sems[b],
                           add=True)
        for b in range(nbuf):
          wait_scatter(b, j + b)
          pltpu.async_copy(table_hbm.at[src_v.at[j + nbuf + b]], rows[b],
                           sems[b])
        return carry

      lax.fori_loop(0, CH // nbuf - 1, step, 0, unroll=False)
      j = CH - nbuf
      for b in range(nbuf):
        wait_gather(b, j + b)
        pltpu.async_copy(rows[b], acc_sh.at[dst_v.at[j + b]], sems[b],
                         add=True)
      for b in range(nbuf):
        wait_scatter(b, j + b)

    plsc.subcore_barrier()
    pltpu.sync_copy(acc_sh.at[pl.ds(sid * RPT, RPT)],
                    out_hbm.at[cid, pl.ds(sid * RPT, RPT)])

  return edge_pass


_edge_pass_16 = _make_edge_pass(16, const_table=True)
_edge_pass_32 = _make_edge_pass(32, const_table=False)


BN = 1024


def _prep_body(deg_ref, x_ref, w1_ref, dinv_ref, s_ref):
  deg = deg_ref[0, :, 0:1] + deg_ref[1, :, 0:1] + 1.0
  dinv = 1.0 / jnp.sqrt(deg)
  dinv_ref[...] = dinv
  hw = jnp.dot(x_ref[...], w1_ref[...], preferred_element_type=jnp.float32)
  s_ref[...] = dinv * hw


def _tc_prep(degp, xpad, w1):
  return pl.pallas_call(
      _prep_body,
      grid=(NP // BN,),
      in_specs=[
          pl.BlockSpec((NCORES, BN, 16), lambda i: (0, i, 0)),
          pl.BlockSpec((BN, D), lambda i: (i, 0)),
          pl.BlockSpec((D, 32), lambda i: (0, 0)),
      ],
      out_specs=[
          pl.BlockSpec((BN, 1), lambda i: (i, 0)),
          pl.BlockSpec((BN, 32), lambda i: (i, 0)),
      ],
      out_shape=[
          jax.ShapeDtypeStruct((NP, 1), jnp.float32),
          jax.ShapeDtypeStruct((NP, 32), jnp.float32),
      ],
  )(degp, xpad, w1)


def _mid_body(p_ref, s_ref, dinv_ref, b_ref, wn_ref, h_ref, sn_ref):
  dinv = dinv_ref[...]
  h = jnp.tanh(dinv * (p_ref[0] + p_ref[1] + s_ref[...]) + b_ref[...])
  h_ref[...] = h
  sn_ref[...] = dinv * jnp.dot(h, wn_ref[...],
                               preferred_element_type=jnp.float32)


def _tc_mid(p, s, dinv, b, wn):
  return pl.pallas_call(
      _mid_body,
      grid=(NP // BN,),
      in_specs=[
          pl.BlockSpec((NCORES, BN, 32), lambda i: (0, i, 0)),
          pl.BlockSpec((BN, 32), lambda i: (i, 0)),
          pl.BlockSpec((BN, 1), lambda i: (i, 0)),
          pl.BlockSpec((1, 32), lambda i: (0, 0)),
          pl.BlockSpec((32, 32), lambda i: (0, 0)),
      ],
      out_specs=[
          pl.BlockSpec((BN, 32), lambda i: (i, 0)),
          pl.BlockSpec((BN, 32), lambda i: (i, 0)),
      ],
      out_shape=[
          jax.ShapeDtypeStruct((NP, 32), jnp.float32),
          jax.ShapeDtypeStruct((NP, 32), jnp.float32),
      ],
  )(p, s, dinv, b, wn)


BNF = 1000


def _final_body(p_ref, s_ref, dinv_ref, b_ref, h1_ref, h2_ref,
                l1w_ref, l1b_ref, l2w_ref, l2b_ref, y_ref,
                logits_ref, feat_ref, loss_ref, acc_ref):
  i = pl.program_id(0)
  dinv = dinv_ref[...]
  h3 = jnp.tanh(dinv * (p_ref[0] + p_ref[1] + s_ref[...]) + b_ref[...])
  cat = jnp.concatenate([h1_ref[...], h2_ref[...], h3], axis=1)
  hidden = jnp.dot(cat, l1w_ref[...],
                   preferred_element_type=jnp.float32) + l1b_ref[...]
  feat_ref[...] = hidden
  hr = jnp.maximum(hidden, 0.0)
  lg = jnp.dot(hr, l2w_ref[...],
               preferred_element_type=jnp.float32) + l2b_ref[...]
  m = jnp.max(lg, axis=1, keepdims=True)
  ls = lg - m - jnp.log(jnp.sum(jnp.exp(lg - m), axis=1, keepdims=True))
  logits_ref[...] = ls
  y2 = y_ref[...]
  l0 = ls[:, 0:1]
  l1 = ls[:, 1:2]
  picked = jnp.where(y2 == 0, l0, l1)
  pred = (l1 > l0).astype(jnp.int32)
  correct = (pred == y2).astype(jnp.float32)

  @pl.when(i == 0)
  def _():
    loss_ref[0, 0] = 0.0
    acc_ref[0, 0] = 0.0

  loss_ref[0, 0] += -jnp.sum(picked) / N
  acc_ref[0, 0] += jnp.sum(correct) / N


def _tc_final(p, s, dinv, b, h1, h2, l1w, l1b, l2w, l2b, y2):
  return pl.pallas_call(
      _final_body,
      grid=(N // BNF,),
      in_specs=[
          pl.BlockSpec((NCORES, BNF, 32), lambda i: (0, i, 0)),
          pl.BlockSpec((BNF, 32), lambda i: (i, 0)),
          pl.BlockSpec((BNF, 1), lambda i: (i, 0)),
          pl.BlockSpec((1, 32), lambda i: (0, 0)),
          pl.BlockSpec((BNF, 32), lambda i: (i, 0)),
          pl.BlockSpec((BNF, 32), lambda i: (i, 0)),
          pl.BlockSpec((96, 128), lambda i: (0, 0)),
          pl.BlockSpec((1, 128), lambda i: (0, 0)),
          pl.BlockSpec((128, NUM_CLASSES), lambda i: (0, 0)),
          pl.BlockSpec((1, NUM_CLASSES), lambda i: (0, 0)),
          pl.BlockSpec((BNF, 1), lambda i: (i, 0)),
      ],
      out_specs=[
          pl.BlockSpec((BNF, NUM_CLASSES), lambda i: (i, 0)),
          pl.BlockSpec((BNF, 128), lambda i: (i, 0)),
          pl.BlockSpec((1, 1), lambda i: (0, 0), memory_space=pltpu.SMEM),
          pl.BlockSpec((1, 1), lambda i: (0, 0), memory_space=pltpu.SMEM),
      ],
      out_shape=[
          jax.ShapeDtypeStruct((N, NUM_CLASSES), jnp.float32),
          jax.ShapeDtypeStruct((N, 128), jnp.float32),
          jax.ShapeDtypeStruct((1, 1), jnp.float32),
          jax.ShapeDtypeStruct((1, 1), jnp.float32),
      ],
  )(p, s, dinv, b, h1, h2, l1w, l1b, l2w, l2b, y2)


def kernel(x, edge_index, batch, y, W1, b1, W2, b2, W3, b3,
           lin1_W, lin1_b, lin2_W, lin2_b):
  del batch
  src = edge_index[0]
  dst = edge_index[1]
  pad = jnp.full((EPAD - E,), N, dtype=jnp.int32)
  srcp = jnp.concatenate([src, pad]).reshape(NWORK * CH, C)
  dstp = jnp.concatenate([dst, pad]).reshape(NWORK * CH, C)
  xpad = jnp.concatenate(
      [x, jnp.zeros((NP - N, D), dtype=jnp.float32)], axis=0)
  ones16 = jnp.ones((C, 16), dtype=jnp.float32)
  zeros16 = jnp.zeros((NP, 16), dtype=jnp.float32)
  zeros32 = jnp.zeros((NP, 32), dtype=jnp.float32)

  degp = _edge_pass_16(srcp, dstp, ones16, zeros16)
  dinv, s1 = _tc_prep(degp, xpad, W1)
  p1 = _edge_pass_32(srcp, dstp, s1, zeros32)
  h1, s2 = _tc_mid(p1, s1, dinv, b1.reshape(1, 32), W2)
  p2 = _edge_pass_32(srcp, dstp, s2, zeros32)
  h2, s3 = _tc_mid(p2, s2, dinv, b2.reshape(1, 32), W3)
  p3 = _edge_pass_32(srcp, dstp, s3, zeros32)
  logits, feature, loss, acc = _tc_final(
      p3, s3, dinv, b3.reshape(1, 32), h1, h2,
      lin1_W, lin1_b.reshape(1, 128), lin2_W, lin2_b.reshape(1, NUM_CLASSES),
      y.reshape(N, 1).astype(jnp.int32))
  return logits, loss.reshape(()), acc.reshape(()), feature


# speedup vs baseline: 1.7479x; 1.7479x over previous
import functools

import jax
import jax.numpy as jnp
from jax import lax
from jax.experimental import pallas as pl
from jax.experimental.pallas import tpu as pltpu
from jax.experimental.pallas import tpu_sc as plsc

N = 10000
E = 160000
D = 256
NUM_CLASSES = 2

NCORES = 2
NSUB = 16
NWORK = NCORES * NSUB
C = 125
CH = 40
NP = N
RPT = NP // NSUB


def _make_edge_pass(width, const_table):
  mesh = plsc.VectorSubcoreMesh(core_axis_name="c", subcore_axis_name="s")

  nbuf = 4

  @functools.partial(
      pl.kernel,
      out_type=jax.ShapeDtypeStruct((NCORES, NP, width), jnp.float32),
      mesh=mesh,
      scratch_types=[
          pltpu.VMEM((CH, C), jnp.int32),
          pltpu.VMEM((CH, C), jnp.int32),
          [pltpu.VMEM((C, width), jnp.float32) for _ in range(nbuf)],
          pltpu.VMEM_SHARED((NP, width), jnp.float32),
          [pltpu.SemaphoreType.DMA for _ in range(nbuf)],
      ],
      compiler_params=pltpu.CompilerParams(use_tc_tiling_on_sc=False),
  )
  def edge_pass(src_hbm, dst_hbm, table_hbm, zeros_hbm, out_hbm,
                src_v, dst_v, rows, acc_sh, sems):
    cid = lax.axis_index("c")
    sid = lax.axis_index("s")
    wid = cid * NSUB + sid
    pltpu.sync_copy(zeros_hbm.at[pl.ds(sid * RPT, RPT)],
                    acc_sh.at[pl.ds(sid * RPT, RPT)])
    pltpu.sync_copy(src_hbm.at[pl.ds(wid * CH, CH)], src_v)
    pltpu.sync_copy(dst_hbm.at[pl.ds(wid * CH, CH)], dst_v)
    plsc.subcore_barrier()

    def wait_gather(b, j):
      pltpu.make_async_copy(table_hbm.at[src_v.at[j]], rows[b],
                            sems[b]).wait()

    def wait_scatter(b, j):
      pltpu.make_async_copy(rows[b], acc_sh.at[dst_v.at[j]],
                            sems[b]).wait()

    if const_table:
      pltpu.sync_copy(table_hbm, rows[0])
      for b in range(nbuf):
        pltpu.async_copy(rows[0], acc_sh.at[dst_v.at[b]], sems[b],
                         add=True)

      def step(jj, carry):
        j = jj * nbuf
        for b in range(nbuf):
          pltpu.make_async_copy(rows[0], acc_sh.at[dst_v.at[j + b]],
                                sems[b]).wait()
          pltpu.async_copy(rows[0], acc_sh.at[dst_v.at[j + b]], sems[b],
                           add=True)
        return carry

      lax.fori_loop(1, CH // nbuf, step, 0, unroll=False)
      for b in range(nbuf):
        pltpu.make_async_copy(rows[0], acc_sh.at[dst_v.at[CH - nbuf + b]],
                              sems[b]).wait()
    else:
      for b in range(nbuf):
        pltpu.async_copy(table_hbm.at[src_v.at[b]], rows[b], sems[b])

      def step(jj, carry):
        j = jj * nbuf
        for b in range(nbuf):
          wait_gather(b, j + b)
          pltpu.async_copy(rows[b], acc_sh.at[dst_v.at[j + b]], sems[b],
                           add=True)
        for b in range(nbuf):
          wait_scatter(b, j + b)
          pltpu.async_copy(table_hbm.at[src_v.at[j + nbuf + b]], rows[b],
                           sems[b])
        return carry

      lax.fori_loop(0, CH // nbuf - 1, step, 0, unroll=False)
      j = CH - nbuf
      for b in range(nbuf):
        wait_gather(b, j + b)
        pltpu.async_copy(rows[b], acc_sh.at[dst_v.at[j + b]], sems[b],
                         add=True)
      for b in range(nbuf):
        wait_scatter(b, j + b)

    plsc.subcore_barrier()
    pltpu.sync_copy(acc_sh.at[pl.ds(sid * RPT, RPT)],
                    out_hbm.at[cid, pl.ds(sid * RPT, RPT)])

  return edge_pass


_edge_pass_16 = _make_edge_pass(16, const_table=True)
_edge_pass_32 = _make_edge_pass(32, const_table=False)


BN = 1000


def _prep_body(deg_ref, x_ref, w1_ref, dinv_ref, s_ref):
  deg = deg_ref[0, :, 0:1] + deg_ref[1, :, 0:1] + 1.0
  dinv = 1.0 / jnp.sqrt(deg)
  dinv_ref[...] = dinv
  hw = jnp.dot(x_ref[...], w1_ref[...], preferred_element_type=jnp.float32)
  s_ref[...] = dinv * hw


def _tc_prep(degp, xpad, w1):
  return pl.pallas_call(
      _prep_body,
      grid=(NP // BN,),
      in_specs=[
          pl.BlockSpec((NCORES, BN, 16), lambda i: (0, i, 0)),
          pl.BlockSpec((BN, D), lambda i: (i, 0)),
          pl.BlockSpec((D, 32), lambda i: (0, 0)),
      ],
      out_specs=[
          pl.BlockSpec((BN, 1), lambda i: (i, 0)),
          pl.BlockSpec((BN, 32), lambda i: (i, 0)),
      ],
      out_shape=[
          jax.ShapeDtypeStruct((NP, 1), jnp.float32),
          jax.ShapeDtypeStruct((NP, 32), jnp.float32),
      ],
  )(degp, xpad, w1)


def _mid_body(p_ref, s_ref, dinv_ref, b_ref, wn_ref, h_ref, sn_ref):
  dinv = dinv_ref[...]
  h = jnp.tanh(dinv * (p_ref[0] + p_ref[1] + s_ref[...]) + b_ref[...])
  h_ref[...] = h
  sn_ref[...] = dinv * jnp.dot(h, wn_ref[...],
                               preferred_element_type=jnp.float32)


def _tc_mid(p, s, dinv, b, wn):
  return pl.pallas_call(
      _mid_body,
      grid=(NP // BN,),
      in_specs=[
          pl.BlockSpec((NCORES, BN, 32), lambda i: (0, i, 0)),
          pl.BlockSpec((BN, 32), lambda i: (i, 0)),
          pl.BlockSpec((BN, 1), lambda i: (i, 0)),
          pl.BlockSpec((1, 32), lambda i: (0, 0)),
          pl.BlockSpec((32, 32), lambda i: (0, 0)),
      ],
      out_specs=[
          pl.BlockSpec((BN, 32), lambda i: (i, 0)),
          pl.BlockSpec((BN, 32), lambda i: (i, 0)),
      ],
      out_shape=[
          jax.ShapeDtypeStruct((NP, 32), jnp.float32),
          jax.ShapeDtypeStruct((NP, 32), jnp.float32),
      ],
  )(p, s, dinv, b, wn)


BNF = 1000


def _final_body(p_ref, s_ref, dinv_ref, b_ref, h1_ref, h2_ref,
                l1w_ref, l1b_ref, l2w_ref, l2b_ref, y_ref,
                logits_ref, feat_ref, loss_ref, acc_ref):
  i = pl.program_id(0)
  dinv = dinv_ref[...]
  h3 = jnp.tanh(dinv * (p_ref[0] + p_ref[1] + s_ref[...]) + b_ref[...])
  cat = jnp.concatenate([h1_ref[...], h2_ref[...], h3], axis=1)
  hidden = jnp.dot(cat, l1w_ref[...],
                   preferred_element_type=jnp.float32) + l1b_ref[...]
  feat_ref[...] = hidden
  hr = jnp.maximum(hidden, 0.0)
  lg = jnp.dot(hr, l2w_ref[...],
               preferred_element_type=jnp.float32) + l2b_ref[...]
  m = jnp.max(lg, axis=1, keepdims=True)
  ls = lg - m - jnp.log(jnp.sum(jnp.exp(lg - m), axis=1, keepdims=True))
  logits_ref[...] = ls
  y2 = y_ref[...]
  l0 = ls[:, 0:1]
  l1 = ls[:, 1:2]
  picked = jnp.where(y2 == 0, l0, l1)
  pred = (l1 > l0).astype(jnp.int32)
  correct = (pred == y2).astype(jnp.float32)

  @pl.when(i == 0)
  def _():
    loss_ref[0, 0] = 0.0
    acc_ref[0, 0] = 0.0

  loss_ref[0, 0] += -jnp.sum(picked) / N
  acc_ref[0, 0] += jnp.sum(correct) / N


def _tc_final(p, s, dinv, b, h1, h2, l1w, l1b, l2w, l2b, y2):
  return pl.pallas_call(
      _final_body,
      grid=(N // BNF,),
      in_specs=[
          pl.BlockSpec((NCORES, BNF, 32), lambda i: (0, i, 0)),
          pl.BlockSpec((BNF, 32), lambda i: (i, 0)),
          pl.BlockSpec((BNF, 1), lambda i: (i, 0)),
          pl.BlockSpec((1, 32), lambda i: (0, 0)),
          pl.BlockSpec((BNF, 32), lambda i: (i, 0)),
          pl.BlockSpec((BNF, 32), lambda i: (i, 0)),
          pl.BlockSpec((96, 128), lambda i: (0, 0)),
          pl.BlockSpec((1, 128), lambda i: (0, 0)),
          pl.BlockSpec((128, NUM_CLASSES), lambda i: (0, 0)),
          pl.BlockSpec((1, NUM_CLASSES), lambda i: (0, 0)),
          pl.BlockSpec((BNF, 1), lambda i: (i, 0)),
      ],
      out_specs=[
          pl.BlockSpec((BNF, NUM_CLASSES), lambda i: (i, 0)),
          pl.BlockSpec((BNF, 128), lambda i: (i, 0)),
          pl.BlockSpec((1, 1), lambda i: (0, 0), memory_space=pltpu.SMEM),
          pl.BlockSpec((1, 1), lambda i: (0, 0), memory_space=pltpu.SMEM),
      ],
      out_shape=[
          jax.ShapeDtypeStruct((N, NUM_CLASSES), jnp.float32),
          jax.ShapeDtypeStruct((N, 128), jnp.float32),
          jax.ShapeDtypeStruct((1, 1), jnp.float32),
          jax.ShapeDtypeStruct((1, 1), jnp.float32),
      ],
  )(p, s, dinv, b, h1, h2, l1w, l1b, l2w, l2b, y2)


def kernel(x, edge_index, batch, y, W1, b1, W2, b2, W3, b3,
           lin1_W, lin1_b, lin2_W, lin2_b):
  del batch
  srcp = edge_index[0].reshape(NWORK * CH, C)
  dstp = edge_index[1].reshape(NWORK * CH, C)
  ones16 = jnp.ones((C, 16), dtype=jnp.float32)
  zeros16 = jnp.zeros((NP, 16), dtype=jnp.float32)
  zeros32 = jnp.zeros((NP, 32), dtype=jnp.float32)

  degp = _edge_pass_16(srcp, dstp, ones16, zeros16)
  dinv, s1 = _tc_prep(degp, x, W1)
  p1 = _edge_pass_32(srcp, dstp, s1, zeros32)
  h1, s2 = _tc_mid(p1, s1, dinv, b1.reshape(1, 32), W2)
  p2 = _edge_pass_32(srcp, dstp, s2, zeros32)
  h2, s3 = _tc_mid(p2, s2, dinv, b2.reshape(1, 32), W3)
  p3 = _edge_pass_32(srcp, dstp, s3, zeros32)
  logits, feature, loss, acc = _tc_final(
      p3, s3, dinv, b3.reshape(1, 32), h1, h2,
      lin1_W, lin1_b.reshape(1, 128), lin2_W, lin2_b.reshape(1, NUM_CLASSES),
      y.reshape(N, 1).astype(jnp.int32))
  return logits, loss.reshape(()), acc.reshape(()), feature


# speedup vs baseline: 1.8144x; 1.0380x over previous
import functools

import jax
import jax.numpy as jnp
from jax import lax
from jax.experimental import pallas as pl
from jax.experimental.pallas import tpu as pltpu
from jax.experimental.pallas import tpu_sc as plsc

N = 10000
E = 160000
D = 256
NUM_CLASSES = 2

NCORES = 2
NSUB = 16
NWORK = NCORES * NSUB
C = 125
CH = 40
NP = N
RPT = NP // NSUB


def _make_edge_pass(width, const_table):
  mesh = plsc.VectorSubcoreMesh(core_axis_name="c", subcore_axis_name="s")

  nbuf = 4

  @functools.partial(
      pl.kernel,
      out_type=jax.ShapeDtypeStruct((NCORES, NP, width), jnp.float32),
      mesh=mesh,
      scratch_types=[
          pltpu.VMEM((CH, C), jnp.int32),
          pltpu.VMEM((CH, C), jnp.int32),
          [pltpu.VMEM((C, width), jnp.float32) for _ in range(nbuf)],
          pltpu.VMEM_SHARED((NP, width), jnp.float32),
          [pltpu.SemaphoreType.DMA for _ in range(nbuf)],
      ],
      compiler_params=pltpu.CompilerParams(use_tc_tiling_on_sc=False),
  )
  def edge_pass(src_hbm, dst_hbm, table_hbm, zeros_hbm, out_hbm,
                src_v, dst_v, rows, acc_sh, sems):
    cid = lax.axis_index("c")
    sid = lax.axis_index("s")
    wid = cid * NSUB + sid
    pltpu.sync_copy(zeros_hbm.at[pl.ds(sid * RPT, RPT)],
                    acc_sh.at[pl.ds(sid * RPT, RPT)])
    pltpu.sync_copy(src_hbm.at[pl.ds(wid * CH, CH)], src_v)
    pltpu.sync_copy(dst_hbm.at[pl.ds(wid * CH, CH)], dst_v)
    plsc.subcore_barrier()

    def wait_gather(b, j):
      pltpu.make_async_copy(table_hbm.at[src_v.at[j]], rows[b],
                            sems[b]).wait()

    def wait_scatter(b, j):
      pltpu.make_async_copy(rows[b], acc_sh.at[dst_v.at[j]],
                            sems[b]).wait()

    if const_table:
      pltpu.sync_copy(table_hbm, rows[0])
      for b in range(nbuf):
        pltpu.async_copy(rows[0], acc_sh.at[dst_v.at[b]], sems[b],
                         add=True)

      def step(jj, carry):
        j = jj * nbuf
        for b in range(nbuf):
          pltpu.make_async_copy(rows[0], acc_sh.at[dst_v.at[j + b]],
                                sems[b]).wait()
          pltpu.async_copy(rows[0], acc_sh.at[dst_v.at[j + b]], sems[b],
                           add=True)
        return carry

      lax.fori_loop(1, CH // nbuf, step, 0, unroll=False)
      for b in range(nbuf):
        pltpu.make_async_copy(rows[0], acc_sh.at[dst_v.at[CH - nbuf + b]],
                              sems[b]).wait()
    else:
      for b in range(nbuf):
        pltpu.async_copy(table_hbm.at[src_v.at[b]], rows[b], sems[b])

      def step(jj, carry):
        j = jj * nbuf
        for b in range(nbuf):
          wait_gather(b, j + b)
          pltpu.async_copy(rows[b], acc_sh.at[dst_v.at[j + b]], sems[b],
                           add=True)
        for b in range(nbuf):
          wait_scatter(b, j + b)
          pltpu.async_copy(table_hbm.at[src_v.at[j + nbuf + b]], rows[b],
                           sems[b])
        return carry

      lax.fori_loop(0, CH // nbuf - 1, step, 0, unroll=False)
      j = CH - nbuf
      for b in range(nbuf):
        wait_gather(b, j + b)
        pltpu.async_copy(rows[b], acc_sh.at[dst_v.at[j + b]], sems[b],
                         add=True)
      for b in range(nbuf):
        wait_scatter(b, j + b)

    plsc.subcore_barrier()
    pltpu.sync_copy(acc_sh.at[pl.ds(sid * RPT, RPT)],
                    out_hbm.at[cid, pl.ds(sid * RPT, RPT)])

  return edge_pass


_edge_pass_16 = _make_edge_pass(16, const_table=True)
_edge_pass_32 = _make_edge_pass(32, const_table=False)


BN = 5000


def _mm1_body(x_ref, w1_ref, u_ref):
  u_ref[...] = jnp.dot(x_ref[...], w1_ref[...],
                       preferred_element_type=jnp.float32)


def _tc_mm1(x, w1):
  return pl.pallas_call(
      _mm1_body,
      grid=(NP // BN,),
      in_specs=[
          pl.BlockSpec((BN, D), lambda i: (i, 0)),
          pl.BlockSpec((D, 32), lambda i: (0, 0)),
      ],
      out_specs=pl.BlockSpec((BN, 32), lambda i: (i, 0)),
      out_shape=jax.ShapeDtypeStruct((NP, 32), jnp.float32),
  )(x, w1)


def _prep_body(deg_ref, u_ref, dinv_ref, s_ref):
  deg = deg_ref[0, :, 0:1] + deg_ref[1, :, 0:1] + 1.0
  dinv = 1.0 / jnp.sqrt(deg)
  dinv_ref[...] = dinv
  s_ref[...] = dinv * u_ref[...]


def _tc_prep(degp, u1):
  return pl.pallas_call(
      _prep_body,
      grid=(NP // BN,),
      in_specs=[
          pl.BlockSpec((NCORES, BN, 16), lambda i: (0, i, 0)),
          pl.BlockSpec((BN, 32), lambda i: (i, 0)),
      ],
      out_specs=[
          pl.BlockSpec((BN, 1), lambda i: (i, 0)),
          pl.BlockSpec((BN, 32), lambda i: (i, 0)),
      ],
      out_shape=[
          jax.ShapeDtypeStruct((NP, 1), jnp.float32),
          jax.ShapeDtypeStruct((NP, 32), jnp.float32),
      ],
  )(degp, u1)


def _mid_body(p_ref, s_ref, dinv_ref, b_ref, wn_ref, h_ref, sn_ref):
  dinv = dinv_ref[...]
  h = jnp.tanh(dinv * (p_ref[0] + p_ref[1] + s_ref[...]) + b_ref[...])
  h_ref[...] = h
  sn_ref[...] = dinv * jnp.dot(h, wn_ref[...],
                               preferred_element_type=jnp.float32)


def _tc_mid(p, s, dinv, b, wn):
  return pl.pallas_call(
      _mid_body,
      grid=(NP // BN,),
      in_specs=[
          pl.BlockSpec((NCORES, BN, 32), lambda i: (0, i, 0)),
          pl.BlockSpec((BN, 32), lambda i: (i, 0)),
          pl.BlockSpec((BN, 1), lambda i: (i, 0)),
          pl.BlockSpec((1, 32), lambda i: (0, 0)),
          pl.BlockSpec((32, 32), lambda i: (0, 0)),
      ],
      out_specs=[
          pl.BlockSpec((BN, 32), lambda i: (i, 0)),
          pl.BlockSpec((BN, 32), lambda i: (i, 0)),
      ],
      out_shape=[
          jax.ShapeDtypeStruct((NP, 32), jnp.float32),
          jax.ShapeDtypeStruct((NP, 32), jnp.float32),
      ],
  )(p, s, dinv, b, wn)


BNF = 5000


def _final_body(p_ref, s_ref, dinv_ref, b_ref, h1_ref, h2_ref,
                l1w_ref, l1b_ref, l2w_ref, l2b_ref, y_ref,
                logits_ref, feat_ref, loss_ref, acc_ref):
  i = pl.program_id(0)
  dinv = dinv_ref[...]
  h3 = jnp.tanh(dinv * (p_ref[0] + p_ref[1] + s_ref[...]) + b_ref[...])
  cat = jnp.concatenate([h1_ref[...], h2_ref[...], h3], axis=1)
  hidden = jnp.dot(cat, l1w_ref[...],
                   preferred_element_type=jnp.float32) + l1b_ref[...]
  feat_ref[...] = hidden
  hr = jnp.maximum(hidden, 0.0)
  lg = jnp.dot(hr, l2w_ref[...],
               preferred_element_type=jnp.float32) + l2b_ref[...]
  m = jnp.max(lg, axis=1, keepdims=True)
  ls = lg - m - jnp.log(jnp.sum(jnp.exp(lg - m), axis=1, keepdims=True))
  logits_ref[...] = ls
  y2 = y_ref[...]
  l0 = ls[:, 0:1]
  l1 = ls[:, 1:2]
  picked = jnp.where(y2 == 0, l0, l1)
  pred = (l1 > l0).astype(jnp.int32)
  correct = (pred == y2).astype(jnp.float32)

  @pl.when(i == 0)
  def _():
    loss_ref[0, 0] = 0.0
    acc_ref[0, 0] = 0.0

  loss_ref[0, 0] += -jnp.sum(picked) / N
  acc_ref[0, 0] += jnp.sum(correct) / N


def _tc_final(p, s, dinv, b, h1, h2, l1w, l1b, l2w, l2b, y2):
  return pl.pallas_call(
      _final_body,
      grid=(N // BNF,),
      in_specs=[
          pl.BlockSpec((NCORES, BNF, 32), lambda i: (0, i, 0)),
          pl.BlockSpec((BNF, 32), lambda i: (i, 0)),
          pl.BlockSpec((BNF, 1), lambda i: (i, 0)),
          pl.BlockSpec((1, 32), lambda i: (0, 0)),
          pl.BlockSpec((BNF, 32), lambda i: (i, 0)),
          pl.BlockSpec((BNF, 32), lambda i: (i, 0)),
          pl.BlockSpec((96, 128), lambda i: (0, 0)),
          pl.BlockSpec((1, 128), lambda i: (0, 0)),
          pl.BlockSpec((128, NUM_CLASSES), lambda i: (0, 0)),
          pl.BlockSpec((1, NUM_CLASSES), lambda i: (0, 0)),
          pl.BlockSpec((BNF, 1), lambda i: (i, 0)),
      ],
      out_specs=[
          pl.BlockSpec((BNF, NUM_CLASSES), lambda i: (i, 0)),
          pl.BlockSpec((BNF, 128), lambda i: (i, 0)),
          pl.BlockSpec((1, 1), lambda i: (0, 0), memory_space=pltpu.SMEM),
          pl.BlockSpec((1, 1), lambda i: (0, 0), memory_space=pltpu.SMEM),
      ],
      out_shape=[
          jax.ShapeDtypeStruct((N, NUM_CLASSES), jnp.float32),
          jax.ShapeDtypeStruct((N, 128), jnp.float32),
          jax.ShapeDtypeStruct((1, 1), jnp.float32),
          jax.ShapeDtypeStruct((1, 1), jnp.float32),
      ],
  )(p, s, dinv, b, h1, h2, l1w, l1b, l2w, l2b, y2)


def kernel(x, edge_index, batch, y, W1, b1, W2, b2, W3, b3,
           lin1_W, lin1_b, lin2_W, lin2_b):
  del batch
  srcp = edge_index[0].reshape(NWORK * CH, C)
  dstp = edge_index[1].reshape(NWORK * CH, C)
  ones16 = jnp.ones((C, 16), dtype=jnp.float32)
  zeros16 = jnp.zeros((NP, 16), dtype=jnp.float32)
  zeros32 = jnp.zeros((NP, 32), dtype=jnp.float32)

  u1 = _tc_mm1(x, W1)
  degp = _edge_pass_16(srcp, dstp, ones16, zeros16)
  dinv, s1 = _tc_prep(degp, u1)
  p1 = _edge_pass_32(srcp, dstp, s1, zeros32)
  h1, s2 = _tc_mid(p1, s1, dinv, b1.reshape(1, 32), W2)
  p2 = _edge_pass_32(srcp, dstp, s2, zeros32)
  h2, s3 = _tc_mid(p2, s2, dinv, b2.reshape(1, 32), W3)
  p3 = _edge_pass_32(srcp, dstp, s3, zeros32)
  logits, feature, loss, acc = _tc_final(
      p3, s3, dinv, b3.reshape(1, 32), h1, h2,
      lin1_W, lin1_b.reshape(1, 128), lin2_W, lin2_b.reshape(1, NUM_CLASSES),
      y.reshape(N, 1).astype(jnp.int32))
  return logits, loss.reshape(()), acc.reshape(()), feature


# speedup vs baseline: 2.0220x; 1.1144x over previous
import functools

import jax
import jax.numpy as jnp
from jax import lax
from jax.experimental import pallas as pl
from jax.experimental.pallas import tpu as pltpu
from jax.experimental.pallas import tpu_sc as plsc

N = 10000
E = 160000
D = 256
NUM_CLASSES = 2

NCORES = 2
NSUB = 16
NWORK = NCORES * NSUB
C = 125
CH = 40
NP = N
RPT = NP // NSUB


def _make_edge_pass(width, const_table):
  mesh = plsc.VectorSubcoreMesh(core_axis_name="c", subcore_axis_name="s")

  nbuf = 4

  @functools.partial(
      pl.kernel,
      out_type=jax.ShapeDtypeStruct((NCORES, NP, width), jnp.float32),
      mesh=mesh,
      scratch_types=[
          pltpu.VMEM((CH, C), jnp.int32),
          pltpu.VMEM((CH, C), jnp.int32),
          [pltpu.VMEM((C, width), jnp.float32) for _ in range(nbuf)],
          pltpu.VMEM_SHARED((NP, width), jnp.float32),
          [pltpu.SemaphoreType.DMA for _ in range(nbuf)],
      ],
      compiler_params=pltpu.CompilerParams(use_tc_tiling_on_sc=False),
  )
  def edge_pass(src_hbm, dst_hbm, table_hbm, zeros_hbm, out_hbm,
                src_v, dst_v, rows, acc_sh, sems):
    cid = lax.axis_index("c")
    sid = lax.axis_index("s")
    wid = cid * NSUB + sid
    pltpu.sync_copy(zeros_hbm.at[pl.ds(sid * RPT, RPT)],
                    acc_sh.at[pl.ds(sid * RPT, RPT)])
    pltpu.sync_copy(src_hbm.at[pl.ds(wid * CH, CH)], src_v)
    pltpu.sync_copy(dst_hbm.at[pl.ds(wid * CH, CH)], dst_v)
    plsc.subcore_barrier()

    def wait_gather(b, j):
      pltpu.make_async_copy(table_hbm.at[src_v.at[j]], rows[b],
                            sems[b]).wait()

    def wait_scatter(b, j):
      pltpu.make_async_copy(rows[b], acc_sh.at[dst_v.at[j]],
                            sems[b]).wait()

    if const_table:
      pltpu.sync_copy(table_hbm, rows[0])
      for b in range(nbuf):
        pltpu.async_copy(rows[0], acc_sh.at[dst_v.at[b]], sems[b],
                         add=True)

      def step(jj, carry):
        j = jj * nbuf
        for b in range(nbuf):
          pltpu.make_async_copy(rows[0], acc_sh.at[dst_v.at[j + b]],
                                sems[b]).wait()
          pltpu.async_copy(rows[0], acc_sh.at[dst_v.at[j + b]], sems[b],
                           add=True)
        return carry

      lax.fori_loop(1, CH // nbuf, step, 0, unroll=False)
      for b in range(nbuf):
        pltpu.make_async_copy(rows[0], acc_sh.at[dst_v.at[CH - nbuf + b]],
                              sems[b]).wait()
    else:
      for b in range(nbuf):
        pltpu.async_copy(table_hbm.at[src_v.at[b]], rows[b], sems[b])

      def step(jj, carry):
        j = jj * nbuf
        for b in range(nbuf):
          wait_gather(b, j + b)
          pltpu.async_copy(rows[b], acc_sh.at[dst_v.at[j + b]], sems[b],
                           add=True)
        for b in range(nbuf):
          wait_scatter(b, j + b)
          pltpu.async_copy(table_hbm.at[src_v.at[j + nbuf + b]], rows[b],
                           sems[b])
        return carry

      lax.fori_loop(0, CH // nbuf - 1, step, 0, unroll=False)
      j = CH - nbuf
      for b in range(nbuf):
        wait_gather(b, j + b)
        pltpu.async_copy(rows[b], acc_sh.at[dst_v.at[j + b]], sems[b],
                         add=True)
      for b in range(nbuf):
        wait_scatter(b, j + b)

    plsc.subcore_barrier()
    pltpu.sync_copy(acc_sh.at[pl.ds(sid * RPT, RPT)],
                    out_hbm.at[cid, pl.ds(sid * RPT, RPT)])

  return edge_pass


_edge_deg = _make_edge_pass(32, const_table=True)
_edge_pass_32 = _make_edge_pass(32, const_table=False)


N4 = N // 4
BV = N4


def _mm1_body(x_ref, w1_ref, u_ref):
  u_ref[...] = jnp.dot(x_ref[...], w1_ref[...],
                       preferred_element_type=jnp.float32)


def _tc_mm1(x, w1):
  return pl.pallas_call(
      _mm1_body,
      grid=(2,),
      in_specs=[
          pl.BlockSpec((N // 2, D), lambda i: (i, 0)),
          pl.BlockSpec((D, 32), lambda i: (0, 0)),
      ],
      out_specs=pl.BlockSpec((N // 2, 32), lambda i: (i, 0)),
      out_shape=jax.ShapeDtypeStruct((N, 32), jnp.float32),
  )(x, w1)


def _prep_body(deg_ref, u_ref, dinv_ref, s_ref):
  deg = deg_ref[0] + deg_ref[1] + 1.0
  dinv = 1.0 / jnp.sqrt(deg)
  dinv_ref[...] = dinv
  s_ref[...] = dinv * u_ref[...]


def _tc_prep(degp128, u128):
  return pl.pallas_call(
      _prep_body,
      grid=(1,),
      in_specs=[
          pl.BlockSpec((NCORES, BV, 128), lambda i: (0, i, 0)),
          pl.BlockSpec((BV, 128), lambda i: (i, 0)),
      ],
      out_specs=[
          pl.BlockSpec((BV, 128), lambda i: (i, 0)),
          pl.BlockSpec((BV, 128), lambda i: (i, 0)),
      ],
      out_shape=[
          jax.ShapeDtypeStruct((N4, 128), jnp.float32),
          jax.ShapeDtypeStruct((N4, 128), jnp.float32),
      ],
  )(degp128, u128)


def _mid_body(p_ref, s_ref, dinv_ref, b_ref, w4_ref, h_ref, sn_ref):
  dinv = dinv_ref[...]
  h = jnp.tanh(dinv * (p_ref[0] + p_ref[1] + s_ref[...]) + b_ref[...])
  h_ref[...] = h
  sn_ref[...] = dinv * jnp.dot(h, w4_ref[...],
                               preferred_element_type=jnp.float32)


def _tc_mid(p128, s128, dinv128, b128, w4):
  return pl.pallas_call(
      _mid_body,
      grid=(1,),
      in_specs=[
          pl.BlockSpec((NCORES, BV, 128), lambda i: (0, i, 0)),
          pl.BlockSpec((BV, 128), lambda i: (i, 0)),
          pl.BlockSpec((BV, 128), lambda i: (i, 0)),
          pl.BlockSpec((1, 128), lambda i: (0, 0)),
          pl.BlockSpec((128, 128), lambda i: (0, 0)),
      ],
      out_specs=[
          pl.BlockSpec((BV, 128), lambda i: (i, 0)),
          pl.BlockSpec((BV, 128), lambda i: (i, 0)),
      ],
      out_shape=[
          jax.ShapeDtypeStruct((N4, 128), jnp.float32),
          jax.ShapeDtypeStruct((N4, 128), jnp.float32),
      ],
  )(p128, s128, dinv128, b128, w4)


def _final_body(p_ref, s_ref, dinv_ref, b_ref, h1_ref, h2_ref,
                s1w_ref, s2w_ref, s3w_ref, l1b_ref, l2w_ref, l2b_ref,
                y_ref, logits_ref, feat_ref, loss_ref, acc_ref):
  i = pl.program_id(0)
  dinv = dinv_ref[...]
  h3 = jnp.tanh(dinv * (p_ref[0] + p_ref[1] + s_ref[...]) + b_ref[...])
  hidden = (
      jnp.dot(h1_ref[...], s1w_ref[...], preferred_element_type=jnp.float32)
      + jnp.dot(h2_ref[...], s2w_ref[...], preferred_element_type=jnp.float32)
      + jnp.dot(h3, s3w_ref[...], preferred_element_type=jnp.float32)
      + l1b_ref[...])
  feat_ref[...] = hidden
  hr = jnp.maximum(hidden, 0.0)

  @pl.when(i == 0)
  def _():
    loss_ref[0, 0] = 0.0
    acc_ref[0, 0] = 0.0

  for k in range(4):
    hk = hr[:, 128 * k:128 * (k + 1)]
    lg = jnp.dot(hk, l2w_ref[...],
                 preferred_element_type=jnp.float32) + l2b_ref[...]
    m = jnp.max(lg, axis=1, keepdims=True)
    ls = lg - m - jnp.log(jnp.sum(jnp.exp(lg - m), axis=1, keepdims=True))
    logits_ref[k] = ls
    y2 = y_ref[:, k:k + 1]
    l0 = ls[:, 0:1]
    l1 = ls[:, 1:2]
    picked = jnp.where(y2 == 0, l0, l1)
    pred = (l1 > l0).astype(jnp.int32)
    correct = (pred == y2).astype(jnp.float32)
    loss_ref[0, 0] += -jnp.sum(picked) / N
    acc_ref[0, 0] += jnp.sum(correct) / N


def _tc_final(p128, s128, dinv128, b128, h1, h2, s1w, s2w, s3w,
              l1b512, l2w, l2b, y4):
  return pl.pallas_call(
      _final_body,
      grid=(1,),
      in_specs=[
          pl.BlockSpec((NCORES, BV, 128), lambda i: (0, i, 0)),
          pl.BlockSpec((BV, 128), lambda i: (i, 0)),
          pl.BlockSpec((BV, 128), lambda i: (i, 0)),
          pl.BlockSpec((1, 128), lambda i: (0, 0)),
          pl.BlockSpec((BV, 128), lambda i: (i, 0)),
          pl.BlockSpec((BV, 128), lambda i: (i, 0)),
          pl.BlockSpec((128, 512), lambda i: (0, 0)),
          pl.BlockSpec((128, 512), lambda i: (0, 0)),
          pl.BlockSpec((128, 512), lambda i: (0, 0)),
          pl.BlockSpec((1, 512), lambda i: (0, 0)),
          pl.BlockSpec((128, NUM_CLASSES), lambda i: (0, 0)),
          pl.BlockSpec((1, NUM_CLASSES), lambda i: (0, 0)),
          pl.BlockSpec((BV, 4), lambda i: (i, 0)),
      ],
      out_specs=[
          pl.BlockSpec((4, BV, NUM_CLASSES), lambda i: (0, i, 0)),
          pl.BlockSpec((BV, 512), lambda i: (i, 0)),
          pl.BlockSpec((1, 1), lambda i: (0, 0), memory_space=pltpu.SMEM),
          pl.BlockSpec((1, 1), lambda i: (0, 0), memory_space=pltpu.SMEM),
      ],
      out_shape=[
          jax.ShapeDtypeStruct((4, N4, NUM_CLASSES), jnp.float32),
          jax.ShapeDtypeStruct((N4, 512), jnp.float32),
          jax.ShapeDtypeStruct((1, 1), jnp.float32),
          jax.ShapeDtypeStruct((1, 1), jnp.float32),
      ],
  )(p128, s128, dinv128, b128, h1, h2, s1w, s2w, s3w, l1b512, l2w, l2b, y4)


def kernel(x, edge_index, batch, y, W1, b1, W2, b2, W3, b3,
           lin1_W, lin1_b, lin2_W, lin2_b):
  del batch
  f32 = jnp.float32
  srcp = edge_index[0].reshape(NWORK * CH, C)
  dstp = edge_index[1].reshape(NWORK * CH, C)
  ones32 = jnp.ones((C, 32), dtype=f32)
  zeros32 = jnp.zeros((NP, 32), dtype=f32)
  eye4 = jnp.eye(4, dtype=f32)
  w4_2 = jnp.kron(eye4, W2)
  w4_3 = jnp.kron(eye4, W3)
  s1w = jnp.kron(eye4, lin1_W[0:32, :])
  s2w = jnp.kron(eye4, lin1_W[32:64, :])
  s3w = jnp.kron(eye4, lin1_W[64:96, :])

  u1 = _tc_mm1(x, W1)
  u1_128 = u1.reshape(N4, 128)
  degp = _edge_deg(srcp, dstp, ones32, zeros32)
  degp128 = degp.reshape(NCORES, N4, 128)
  dinv128, s1 = _tc_prep(degp128, u1_128)
  p1 = _edge_pass_32(srcp, dstp, s1.reshape(N, 32), zeros32)
  h1, s2 = _tc_mid(p1.reshape(NCORES, N4, 128), s1, dinv128,
                   jnp.tile(b1, 4).reshape(1, 128), w4_2)
  p2 = _edge_pass_32(srcp, dstp, s2.reshape(N, 32), zeros32)
  h2, s3 = _tc_mid(p2.reshape(NCORES, N4, 128), s2, dinv128,
                   jnp.tile(b2, 4).reshape(1, 128), w4_3)
  p3 = _edge_pass_32(srcp, dstp, s3.reshape(N, 32), zeros32)
  logits4, feat512, loss, acc = _tc_final(
      p3.reshape(NCORES, N4, 128), s3, dinv128,
      jnp.tile(b3, 4).reshape(1, 128), h1, h2, s1w, s2w, s3w,
      jnp.tile(lin1_b, 4).reshape(1, 512), lin2_W,
      lin2_b.reshape(1, NUM_CLASSES), y.reshape(N4, 4).astype(jnp.int32))
  logits = jnp.transpose(logits4, (1, 0, 2)).reshape(N, NUM_CLASSES)
  feature = feat512.reshape(N, 128)
  return logits, loss.reshape(()), acc.reshape(()), feature


# speedup vs baseline: 2.1343x; 1.0556x over previous
import functools

import jax
import jax.numpy as jnp
from jax import lax
from jax.experimental import pallas as pl
from jax.experimental.pallas import tpu as pltpu
from jax.experimental.pallas import tpu_sc as plsc

N = 10000
E = 160000
D = 256
NUM_CLASSES = 2

NCORES = 2
NSUB = 16
NWORK = NCORES * NSUB
C = 125
CH = 40
NP = N
RPT = NP // NSUB


def _make_edge_pass(width, const_table):
  mesh = plsc.VectorSubcoreMesh(core_axis_name="c", subcore_axis_name="s")

  nbuf = 4

  @functools.partial(
      pl.kernel,
      out_type=jax.ShapeDtypeStruct((NCORES, NP, width), jnp.float32),
      mesh=mesh,
      scratch_types=[
          pltpu.VMEM((CH, C), jnp.int32),
          pltpu.VMEM((CH, C), jnp.int32),
          [pltpu.VMEM((C, width), jnp.float32) for _ in range(nbuf)],
          pltpu.VMEM_SHARED((NP, width), jnp.float32),
          [pltpu.SemaphoreType.DMA for _ in range(nbuf)],
      ],
      compiler_params=pltpu.CompilerParams(use_tc_tiling_on_sc=False),
  )
  def edge_pass(src_hbm, dst_hbm, table_hbm, zeros_hbm, out_hbm,
                src_v, dst_v, rows, acc_sh, sems):
    cid = lax.axis_index("c")
    sid = lax.axis_index("s")
    wid = cid * NSUB + sid
    pltpu.sync_copy(zeros_hbm.at[pl.ds(sid * RPT, RPT)],
                    acc_sh.at[pl.ds(sid * RPT, RPT)])
    pltpu.sync_copy(src_hbm.at[pl.ds(wid * CH, CH)], src_v)
    pltpu.sync_copy(dst_hbm.at[pl.ds(wid * CH, CH)], dst_v)
    plsc.subcore_barrier()

    def wait_gather(b, j):
      pltpu.make_async_copy(table_hbm.at[src_v.at[j]], rows[b],
                            sems[b]).wait()

    def wait_scatter(b, j):
      pltpu.make_async_copy(rows[b], acc_sh.at[dst_v.at[j]],
                            sems[b]).wait()

    if const_table:
      pltpu.sync_copy(table_hbm, rows[0])
      for b in range(nbuf):
        pltpu.async_copy(rows[0], acc_sh.at[dst_v.at[b]], sems[b],
                         add=True)

      def step(jj, carry):
        j = jj * nbuf
        for b in range(nbuf):
          pltpu.make_async_copy(rows[0], acc_sh.at[dst_v.at[j + b]],
                                sems[b]).wait()
          pltpu.async_copy(rows[0], acc_sh.at[dst_v.at[j + b]], sems[b],
                           add=True)
        return carry

      lax.fori_loop(1, CH // nbuf, step, 0, unroll=False)
      for b in range(nbuf):
        pltpu.make_async_copy(rows[0], acc_sh.at[dst_v.at[CH - nbuf + b]],
                              sems[b]).wait()
    else:
      for b in range(nbuf):
        pltpu.async_copy(table_hbm.at[src_v.at[b]], rows[b], sems[b])

      def step(jj, carry):
        j = jj * nbuf
        for b in range(nbuf):
          wait_gather(b, j + b)
          pltpu.async_copy(rows[b], acc_sh.at[dst_v.at[j + b]], sems[b],
                           add=True)
        for b in range(nbuf):
          wait_scatter(b, j + b)
          pltpu.async_copy(table_hbm.at[src_v.at[j + nbuf + b]], rows[b],
                           sems[b])
        return carry

      lax.fori_loop(0, CH // nbuf - 1, step, 0, unroll=False)
      j = CH - nbuf
      for b in range(nbuf):
        wait_gather(b, j + b)
        pltpu.async_copy(rows[b], acc_sh.at[dst_v.at[j + b]], sems[b],
                         add=True)
      for b in range(nbuf):
        wait_scatter(b, j + b)

    plsc.subcore_barrier()
    pltpu.sync_copy(acc_sh.at[pl.ds(sid * RPT, RPT)],
                    out_hbm.at[cid, pl.ds(sid * RPT, RPT)])

  return edge_pass


_edge_deg = _make_edge_pass(32, const_table=True)
_edge_pass_32 = _make_edge_pass(32, const_table=False)


N4 = N // 4
BV = N4


def _mm1_body(x_ref, w1_ref, u_ref):
  u_ref[...] = jnp.dot(x_ref[...], w1_ref[...],
                       preferred_element_type=jnp.float32)


def _tc_mm1(x, w1):
  return pl.pallas_call(
      _mm1_body,
      grid=(2,),
      in_specs=[
          pl.BlockSpec((N // 2, D), lambda i: (i, 0)),
          pl.BlockSpec((D, 32), lambda i: (0, 0)),
      ],
      out_specs=pl.BlockSpec((N // 2, 32), lambda i: (i, 0)),
      out_shape=jax.ShapeDtypeStruct((N, 32), jnp.float32),
  )(x, w1)


def _prep_body(deg_ref, u_ref, dinv_ref, s_ref):
  deg = deg_ref[0] + deg_ref[1] + 1.0
  dinv = 1.0 / jnp.sqrt(deg)
  dinv_ref[...] = dinv
  s_ref[...] = dinv * u_ref[...]


def _tc_prep(degp128, u128):
  return pl.pallas_call(
      _prep_body,
      grid=(1,),
      in_specs=[
          pl.BlockSpec((NCORES, BV, 128), lambda i: (0, i, 0)),
          pl.BlockSpec((BV, 128), lambda i: (i, 0)),
      ],
      out_specs=[
          pl.BlockSpec((BV, 128), lambda i: (i, 0)),
          pl.BlockSpec((BV, 128), lambda i: (i, 0)),
      ],
      out_shape=[
          jax.ShapeDtypeStruct((N4, 128), jnp.float32),
          jax.ShapeDtypeStruct((N4, 128), jnp.float32),
      ],
  )(degp128, u128)


def _mid_body(p_ref, s_ref, dinv_ref, b_ref, w4_ref, h_ref, sn_ref):
  dinv = dinv_ref[...]
  h = jnp.tanh(dinv * (p_ref[0] + p_ref[1] + s_ref[...]) + b_ref[...])
  h_ref[...] = h
  sn_ref[...] = dinv * jnp.dot(h, w4_ref[...],
                               preferred_element_type=jnp.float32)


def _tc_mid(p128, s128, dinv128, b128, w4):
  return pl.pallas_call(
      _mid_body,
      grid=(1,),
      in_specs=[
          pl.BlockSpec((NCORES, BV, 128), lambda i: (0, i, 0)),
          pl.BlockSpec((BV, 128), lambda i: (i, 0)),
          pl.BlockSpec((BV, 128), lambda i: (i, 0)),
          pl.BlockSpec((1, 128), lambda i: (0, 0)),
          pl.BlockSpec((128, 128), lambda i: (0, 0)),
      ],
      out_specs=[
          pl.BlockSpec((BV, 128), lambda i: (i, 0)),
          pl.BlockSpec((BV, 128), lambda i: (i, 0)),
      ],
      out_shape=[
          jax.ShapeDtypeStruct((N4, 128), jnp.float32),
          jax.ShapeDtypeStruct((N4, 128), jnp.float32),
      ],
  )(p128, s128, dinv128, b128, w4)


def _final_body(p_ref, s_ref, deg_ref, b_ref, h1_ref, h2_ref,
                l1w_ref, l1b_ref, l2w_ref, l2b_ref, y_ref,
                logits_ref, feat_ref, loss_ref, acc_ref):
  i = pl.program_id(0)
  deg = deg_ref[0, :, 0:1] + deg_ref[1, :, 0:1] + 1.0
  dinv = 1.0 / jnp.sqrt(deg)
  h3 = jnp.tanh(dinv * (p_ref[0] + p_ref[1] + s_ref[...]) + b_ref[...])
  cat = jnp.concatenate([h1_ref[...], h2_ref[...], h3], axis=1)
  hidden = jnp.dot(cat, l1w_ref[...],
                   preferred_element_type=jnp.float32) + l1b_ref[...]
  feat_ref[...] = hidden
  hr = jnp.maximum(hidden, 0.0)
  lg = jnp.dot(hr, l2w_ref[...],
               preferred_element_type=jnp.float32) + l2b_ref[...]
  m = jnp.max(lg, axis=1, keepdims=True)
  ls = lg - m - jnp.log(jnp.sum(jnp.exp(lg - m), axis=1, keepdims=True))
  logits_ref[...] = ls
  y2 = y_ref[...]
  l0 = ls[:, 0:1]
  l1 = ls[:, 1:2]
  picked = jnp.where(y2 == 0, l0, l1)
  pred = (l1 > l0).astype(jnp.int32)
  correct = (pred == y2).astype(jnp.float32)

  @pl.when(i == 0)
  def _():
    loss_ref[0, 0] = 0.0
    acc_ref[0, 0] = 0.0

  loss_ref[0, 0] += -jnp.sum(picked) / N
  acc_ref[0, 0] += jnp.sum(correct) / N


BNF = 5000


def _tc_final(p, s, degp, b, h1, h2, l1w, l1b, l2w, l2b, y2):
  return pl.pallas_call(
      _final_body,
      grid=(N // BNF,),
      in_specs=[
          pl.BlockSpec((NCORES, BNF, 32), lambda i: (0, i, 0)),
          pl.BlockSpec((BNF, 32), lambda i: (i, 0)),
          pl.BlockSpec((NCORES, BNF, 32), lambda i: (0, i, 0)),
          pl.BlockSpec((1, 32), lambda i: (0, 0)),
          pl.BlockSpec((BNF, 32), lambda i: (i, 0)),
          pl.BlockSpec((BNF, 32), lambda i: (i, 0)),
          pl.BlockSpec((96, 128), lambda i: (0, 0)),
          pl.BlockSpec((1, 128), lambda i: (0, 0)),
          pl.BlockSpec((128, NUM_CLASSES), lambda i: (0, 0)),
          pl.BlockSpec((1, NUM_CLASSES), lambda i: (0, 0)),
          pl.BlockSpec((BNF, 1), lambda i: (i, 0)),
      ],
      out_specs=[
          pl.BlockSpec((BNF, NUM_CLASSES), lambda i: (i, 0)),
          pl.BlockSpec((BNF, 128), lambda i: (i, 0)),
          pl.BlockSpec((1, 1), lambda i: (0, 0), memory_space=pltpu.SMEM),
          pl.BlockSpec((1, 1), lambda i: (0, 0), memory_space=pltpu.SMEM),
      ],
      out_shape=[
          jax.ShapeDtypeStruct((N, NUM_CLASSES), jnp.float32),
          jax.ShapeDtypeStruct((N, 128), jnp.float32),
          jax.ShapeDtypeStruct((1, 1), jnp.float32),
          jax.ShapeDtypeStruct((1, 1), jnp.float32),
      ],
  )(p, s, degp, b, h1, h2, l1w, l1b, l2w, l2b, y2)


def kernel(x, edge_index, batch, y, W1, b1, W2, b2, W3, b3,
           lin1_W, lin1_b, lin2_W, lin2_b):
  del batch
  f32 = jnp.float32
  srcp = edge_index[0].reshape(NWORK * CH, C)
  dstp = edge_index[1].reshape(NWORK * CH, C)
  ones32 = jnp.ones((C, 32), dtype=f32)
  zeros32 = jnp.zeros((NP, 32), dtype=f32)
  eye4 = jnp.eye(4, dtype=f32)
  w4_2 = jnp.kron(eye4, W2)
  w4_3 = jnp.kron(eye4, W3)

  u1 = _tc_mm1(x, W1)
  u1_128 = u1.reshape(N4, 128)
  degp = _edge_deg(srcp, dstp, ones32, zeros32)
  degp128 = degp.reshape(NCORES, N4, 128)
  dinv128, s1 = _tc_prep(degp128, u1_128)
  p1 = _edge_pass_32(srcp, dstp, s1.reshape(N, 32), zeros32)
  h1, s2 = _tc_mid(p1.reshape(NCORES, N4, 128), s1, dinv128,
                   jnp.tile(b1, 4).reshape(1, 128), w4_2)
  h1n = h1.reshape(N, 32)
  p2 = _edge_pass_32(srcp, dstp, s2.reshape(N, 32), zeros32)
  h2, s3 = _tc_mid(p2.reshape(NCORES, N4, 128), s2, dinv128,
                   jnp.tile(b2, 4).reshape(1, 128), w4_3)
  h2n = h2.reshape(N, 32)
  s3n = s3.reshape(N, 32)
  p3 = _edge_pass_32(srcp, dstp, s3n, zeros32)
  logits, feature, loss, acc = _tc_final(
      p3, s3n, degp, b3.reshape(1, 32), h1n, h2n,
      lin1_W, lin1_b.reshape(1, 128), lin2_W,
      lin2_b.reshape(1, NUM_CLASSES), y.reshape(N, 1).astype(jnp.int32))
  return logits, loss.reshape(()), acc.reshape(()), feature
